# Initial kernel scaffold; baseline (speedup 1.0000x reference)
import jax
import jax.numpy as jnp
from jax.experimental import pallas as pl


def kernel(x, edge_index, batch, W1, b1, W2, b2, W3, b3, Wl, bl):
    raise NotImplementedError("write your pallas kernel here")


# speedup vs baseline: 43.3974x; 43.3974x over previous
import functools

import jax
import jax.numpy as jnp
from jax import lax
from jax.experimental import pallas as pl
from jax.experimental.pallas import tpu as pltpu
from jax.experimental.pallas import tpu_sc as plsc

NC = 2
NS = 16
CHUNK = 128
GROUP = 1024
NB = 5000


def _mesh():
  return plsc.VectorSubcoreMesh(core_axis_name="c", subcore_axis_name="s")


def _make_deg_kernel(groups, rpt, np_rows):
  @functools.partial(
      pl.kernel,
      out_type=jax.ShapeDtypeStruct((NC, np_rows, 8), jnp.float32),
      mesh=_mesh(),
      compiler_params=pltpu.CompilerParams(use_tc_tiling_on_sc=False),
      scratch_types=[
          pltpu.VMEM((8, CHUNK), jnp.int32),
          pltpu.VMEM((CHUNK, 8), jnp.float32),
          pltpu.VMEM_SHARED((np_rows, 8), jnp.float32),
      ],
  )
  def deg_kernel(dst_hbm, ones_hbm, zeros_hbm, out_hbm, dst_v, ones_v, deg_sh):
    c = lax.axis_index("c")
    s = lax.axis_index("s")
    pltpu.sync_copy(ones_hbm, ones_v)
    pltpu.sync_copy(zeros_hbm, deg_sh.at[pl.ds(s * rpt, rpt)])
    plsc.subcore_barrier()
    tile_base = (c * NS + s) * groups * 8

    def body(g, carry):
      crow = tile_base + g * 8
      pltpu.sync_copy(dst_hbm.at[pl.ds(crow, 8)], dst_v)
      for j in range(8):
        pltpu.sync_copy(ones_v, deg_sh.at[dst_v.at[j]], add=True)
      return carry

    lax.fori_loop(0, groups, body, 0)
    plsc.subcore_barrier()
    pltpu.sync_copy(deg_sh.at[pl.ds(s * rpt, rpt)],
                    out_hbm.at[c].at[pl.ds(s * rpt, rpt)])

  return deg_kernel


def _make_edge_kernel(groups, rpt, np_rows):
  @functools.partial(
      pl.kernel,
      out_type=jax.ShapeDtypeStruct((NC, np_rows, 16), jnp.float32),
      mesh=_mesh(),
      compiler_params=pltpu.CompilerParams(use_tc_tiling_on_sc=False),
      scratch_types=[
          pltpu.VMEM((8, CHUNK), jnp.int32),
          pltpu.VMEM((8, CHUNK), jnp.int32),
          pltpu.VMEM((GROUP, 16), jnp.float32),
          pltpu.SemaphoreType.DMA,
          pltpu.VMEM_SHARED((np_rows, 16), jnp.float32),
      ],
  )
  def edge_kernel(src_hbm, dst_hbm, u_hbm, zeros_hbm, out_hbm,
                  src_v, dst_v, rows_v, sem, agg_sh):
    c = lax.axis_index("c")
    s = lax.axis_index("s")
    pltpu.sync_copy(zeros_hbm, agg_sh.at[pl.ds(s * rpt, rpt)])
    plsc.subcore_barrier()
    tile_base = (c * NS + s) * groups * 8

    def body(g, carry):
      crow = tile_base + g * 8
      pltpu.sync_copy(src_hbm.at[pl.ds(crow, 8)], src_v)
      pltpu.sync_copy(dst_hbm.at[pl.ds(crow, 8)], dst_v)
      descs = [
          pltpu.async_copy(u_hbm.at[src_v.at[j]],
                           rows_v.at[pl.ds(j * CHUNK, CHUNK)], sem)
          for j in range(8)
      ]
      for d in descs:
        d.wait()
      for j in range(8):
        pltpu.sync_copy(rows_v.at[pl.ds(j * CHUNK, CHUNK)],
                        agg_sh.at[dst_v.at[j]], add=True)
      return carry

    lax.fori_loop(0, groups, body, 0)
    plsc.subcore_barrier()
    pltpu.sync_copy(agg_sh.at[pl.ds(s * rpt, rpt)],
                    out_hbm.at[c].at[pl.ds(s * rpt, rpt)])

  return edge_kernel


def _prep_body(degp_ref, x_ref, w_ref, dinv_ref, u_ref):
  d = degp_ref[0, :, 0:1] + degp_ref[1, :, 0:1] + 1.0
  dv = lax.rsqrt(d)
  dinv_ref[...] = dv
  u_ref[...] = dv * jnp.dot(x_ref[...], w_ref[...],
                            preferred_element_type=jnp.float32)


def _layer_body(agg_ref, u_ref, dinv_ref, b_ref, w_ref, un_ref):
  t = agg_ref[0] + agg_ref[1] + u_ref[...]
  o = jnp.maximum(dinv_ref[...] * t + b_ref[...], 0.0)
  un_ref[...] = dinv_ref[...] * jnp.dot(o, w_ref[...],
                                        preferred_element_type=jnp.float32)


def _make_final_body(nblk):
  def _final_body(agg_ref, u_ref, dinv_ref, b_ref, batch_ref, wl_ref, bl_ref,
                  out_ref, acc_ref):
    i = pl.program_id(0)
    t = agg_ref[0] + agg_ref[1] + u_ref[...]
    o = jnp.maximum(dinv_ref[...] * t + b_ref[...], 0.0)
    ids = lax.broadcasted_iota(jnp.int32, (1, 64), 1)
    oh = (batch_ref[...] == ids).astype(jnp.float32)
    ext = jnp.concatenate([o, jnp.ones_like(o)], axis=1)
    part = lax.dot_general(oh, ext, (((0,), (0,)), ((), ())),
                           preferred_element_type=jnp.float32)

    @pl.when(i == 0)
    def _():
      acc_ref[...] = jnp.zeros_like(acc_ref)

    acc_ref[...] += part

    @pl.when(i == nblk - 1)
    def _():
      sums = acc_ref[:, :16]
      cnt = acc_ref[:, 16:17]
      pooled = sums / jnp.maximum(cnt, 1.0)
      out_ref[...] = jnp.dot(pooled, wl_ref[...],
                             preferred_element_type=jnp.float32) + bl_ref[...]

  return _final_body


def kernel(x, edge_index, batch, W1, b1, W2, b2, W3, b3, Wl, bl):
  n = x.shape[0]
  e = edge_index.shape[1]
  f32 = jnp.float32

  rpt = ((n + NS - 1) // NS + 7) // 8 * 8
  np_rows = NS * rpt
  groups = (e + 32 * GROUP - 1) // (32 * GROUP)
  e_pad = 32 * GROUP * groups
  pad = e_pad - e

  src = edge_index[0]
  dst = edge_index[1]
  pad_src = jnp.zeros((pad,), jnp.int32)
  pad_dst = n + (jnp.arange(pad, dtype=jnp.int32) % (np_rows - n))
  src_p = jnp.concatenate([src, pad_src]).reshape(e_pad // CHUNK, CHUNK)
  dst_p = jnp.concatenate([dst, pad_dst]).reshape(e_pad // CHUNK, CHUNK)

  ones_col = jnp.ones((CHUNK, 8), f32)
  zeros_col = jnp.zeros((rpt, 8), f32)
  zeros16 = jnp.zeros((rpt, 16), f32)

  deg_kernel = _make_deg_kernel(groups, rpt, np_rows)
  edge_kernel = _make_edge_kernel(groups, rpt, np_rows)

  deg_parts = deg_kernel(dst_p, ones_col, zeros_col)

  nblk = n // NB
  dinv, u = pl.pallas_call(
      _prep_body,
      grid=(nblk,),
      in_specs=[
          pl.BlockSpec((2, NB, 8), lambda i: (0, i, 0)),
          pl.BlockSpec((NB, 16), lambda i: (i, 0)),
          pl.BlockSpec((16, 16), lambda i: (0, 0)),
      ],
      out_specs=[
          pl.BlockSpec((NB, 1), lambda i: (i, 0)),
          pl.BlockSpec((NB, 16), lambda i: (i, 0)),
      ],
      out_shape=[
          jax.ShapeDtypeStruct((n, 1), f32),
          jax.ShapeDtypeStruct((n, 16), f32),
      ],
  )(deg_parts, x, W1)

  layer_call = pl.pallas_call(
      _layer_body,
      grid=(nblk,),
      in_specs=[
          pl.BlockSpec((2, NB, 16), lambda i: (0, i, 0)),
          pl.BlockSpec((NB, 16), lambda i: (i, 0)),
          pl.BlockSpec((NB, 1), lambda i: (i, 0)),
          pl.BlockSpec((1, 16), lambda i: (0, 0)),
          pl.BlockSpec((16, 16), lambda i: (0, 0)),
      ],
      out_specs=pl.BlockSpec((NB, 16), lambda i: (i, 0)),
      out_shape=jax.ShapeDtypeStruct((n, 16), f32),
  )

  agg = edge_kernel(src_p, dst_p, u, zeros16)
  u = layer_call(agg, u, dinv, b1.reshape(1, 16), W2)
  agg = edge_kernel(src_p, dst_p, u, zeros16)
  u = layer_call(agg, u, dinv, b2.reshape(1, 16), W3)
  agg = edge_kernel(src_p, dst_p, u, zeros16)

  out = pl.pallas_call(
      _make_final_body(nblk),
      grid=(nblk,),
      in_specs=[
          pl.BlockSpec((2, NB, 16), lambda i: (0, i, 0)),
          pl.BlockSpec((NB, 16), lambda i: (i, 0)),
          pl.BlockSpec((NB, 1), lambda i: (i, 0)),
          pl.BlockSpec((1, 16), lambda i: (0, 0)),
          pl.BlockSpec((NB, 1), lambda i: (i, 0)),
          pl.BlockSpec((16, 2), lambda i: (0, 0)),
          pl.BlockSpec((1, 2), lambda i: (0, 0)),
      ],
      out_specs=pl.BlockSpec((64, 2), lambda i: (0, 0)),
      out_shape=jax.ShapeDtypeStruct((64, 2), f32),
      scratch_shapes=[pltpu.VMEM((64, 32), f32)],
  )(agg, u, dinv, b3.reshape(1, 16), batch.reshape(n, 1), Wl,
    bl.reshape(1, 2))

  return out
